# baseline (speedup 1.0000x reference)
import jax
import jax.numpy as jnp
import numpy as np
from jax.experimental import pallas as pl
from jax.experimental.pallas import tpu as pltpu


def _interp_matrix_align_corners(dst, src):
    m = np.zeros((dst, src), np.float32)
    if dst == 1:
        pos = np.zeros((1,), np.float64)
    else:
        pos = np.arange(dst, dtype=np.float64) * (src - 1) / (dst - 1)
    lo = np.clip(np.floor(pos).astype(np.int64), 0, src - 1)
    hi = np.minimum(lo + 1, src - 1)
    frac = (pos - lo).astype(np.float32)
    m[np.arange(dst), lo] += 1.0 - frac
    m[np.arange(dst), hi] += frac
    return m


def _make_kernel(Ws, Ps, Pc, Cc, Cs, Cd, ne, nchain):
    bf16 = jnp.bfloat16
    P2 = ne * Ps
    OB_S1, OB_S2, OB_F1, OB_F2 = 0, Cs, 2 * Cs, 2 * Cs + Cd

    def gelu(x):
        h = 0.5 * x
        return h + h * jnp.tanh(0.7978845608028654 *
                                (x + 0.044715 * (x * x * x)))

    def body(cur_ref, skip_ref, m2t_ref, w1_ref, w2_ref, wr_ref, w3_ref,
             w4_ref, b_ref, out_ref):
        f32 = jnp.float32
        col = jax.lax.broadcasted_iota(jnp.int32, (1, P2), 1) % Ws
        mx_m = (col >= 1).astype(bf16)
        mx_p = (col < Ws - 1).astype(bf16)
        zrow = jnp.zeros((max(Cs, Cd), Ws), f32)

        def shift_dy(z, cout, up):
            pieces = []
            for h in range(ne):
                lo = h * Ps
                if up:
                    pieces += [zrow[0:cout], z[:, lo:lo + Ps - Ws]]
                else:
                    pieces += [z[:, lo + Ws:lo + Ps], zrow[0:cout]]
            return jnp.concatenate(pieces, axis=1)

        def conv3x3(xb, cin, wref, cout):
            t_m = jnp.concatenate([xb[:, P2 - 1:], xb[:, :P2 - 1]], axis=1) * mx_m
            t_p = jnp.concatenate([xb[:, 1:], xb[:, :1]], axis=1) * mx_p
            taps = jnp.concatenate([t_m, xb, t_p], axis=0)
            z = jnp.dot(wref[...], taps, preferred_element_type=f32)
            z_m, z_0, z_p = z[0:cout], z[cout:2 * cout], z[2 * cout:3 * cout]
            return shift_dy(z_m, cout, True) + z_0 + shift_dy(z_p, cout, False)

        def bias(off, cout):
            return b_ref[off:off + cout, :]

        for c in range(nchain):
            e0 = c * ne
            skip = jnp.concatenate([skip_ref[e0 + e] for e in range(ne)], axis=1)
            s1 = gelu(conv3x3(skip.astype(bf16), Cs, w1_ref, Cs)
                      + bias(OB_S1, Cs) + skip)
            s2 = gelu(conv3x3(s1.astype(bf16), Cs, w2_ref, Cs)
                      + bias(OB_S2, Cs) + s1)

            cur_all = cur_ref[e0:e0 + ne].reshape(ne * Cc, Pc).astype(bf16)
            cur_up = jnp.dot(cur_all, m2t_ref[...],
                             preferred_element_type=f32)
            cat = jnp.concatenate(
                [jnp.concatenate(
                    [cur_up[e * Cc:(e + 1) * Cc].astype(bf16) for e in range(ne)],
                    axis=1),
                 s2.astype(bf16)], axis=0)
            x = gelu(conv3x3(cat, Cc + Cs, wr_ref, Cd))

            f1 = gelu(conv3x3(x.astype(bf16), Cd, w3_ref, Cd)
                      + bias(OB_F1, Cd) + x)
            f2 = gelu(conv3x3(f1.astype(bf16), Cd, w4_ref, Cd)
                      + bias(OB_F2, Cd) + f1)

            for e in range(ne):
                out_ref[e0 + e] = f2[:, e * Ps:(e + 1) * Ps].astype(out_ref.dtype)

    return body


def _wall(w):
    co, ci = w.shape[3], w.shape[2]
    return jnp.transpose(w, (0, 3, 1, 2)).reshape(3 * co, 3 * ci).astype(jnp.bfloat16)


def kernel(cur_x, skip_x, skip_w1, skip_b1, skip_w2, skip_b2,
           red_w, fus_w1, fus_b1, fus_w2, fus_b2):
    N, Cc, Hc, Wc = cur_x.shape
    _, Cs, Hs, Ws = skip_x.shape
    Cd = red_w.shape[-1]
    Pc, Ps = Hc * Wc, Hs * Ws

    wh = _interp_matrix_align_corners(Hs, Hc)
    ww = _interp_matrix_align_corners(Ws, Wc)
    m2t = jnp.asarray(np.kron(wh, ww).T.astype(np.float32)).astype(jnp.bfloat16)

    b_pack = jnp.concatenate([skip_b1, skip_b2, fus_b1, fus_b2]).reshape(-1, 1)

    cur_flat = cur_x.reshape(N, Cc, Pc)
    skip_flat = skip_x.reshape(N, Cs, Ps)

    ne = 6 if N % 6 == 0 else (4 if N % 4 == 0 else (2 if N % 2 == 0 else 1))
    nchain = 1
    nblk = ne * nchain
    grid_n = N // nblk

    def const_spec(shape):
        return pl.BlockSpec(shape, lambda n: (0,) * len(shape))

    weights = [_wall(skip_w1), _wall(skip_w2), _wall(red_w),
               _wall(fus_w1), _wall(fus_w2)]

    out = pl.pallas_call(
        _make_kernel(Ws, Ps, Pc, Cc, Cs, Cd, ne, nchain),
        out_shape=jax.ShapeDtypeStruct((N, Cd, Ps), cur_x.dtype),
        grid=(grid_n,),
        in_specs=[
            pl.BlockSpec((nblk, Cc, Pc), lambda n: (n, 0, 0)),
            pl.BlockSpec((nblk, Cs, Ps), lambda n: (n, 0, 0)),
            const_spec(m2t.shape),
            const_spec(weights[0].shape),
            const_spec(weights[1].shape),
            const_spec(weights[2].shape),
            const_spec(weights[3].shape),
            const_spec(weights[4].shape),
            const_spec(b_pack.shape),
        ],
        out_specs=pl.BlockSpec((nblk, Cd, Ps), lambda n: (n, 0, 0)),
        compiler_params=pltpu.CompilerParams(
            dimension_semantics=("parallel",)),
    )(cur_flat, skip_flat, m2t, *weights, b_pack)
    return out.reshape(N, Cd, Hs, Ws)


# speedup vs baseline: 1.0220x; 1.0220x over previous
import jax
import jax.numpy as jnp
import numpy as np
from jax.experimental import pallas as pl
from jax.experimental.pallas import tpu as pltpu


def _interp_matrix_align_corners(dst, src):
    m = np.zeros((dst, src), np.float32)
    if dst == 1:
        pos = np.zeros((1,), np.float64)
    else:
        pos = np.arange(dst, dtype=np.float64) * (src - 1) / (dst - 1)
    lo = np.clip(np.floor(pos).astype(np.int64), 0, src - 1)
    hi = np.minimum(lo + 1, src - 1)
    frac = (pos - lo).astype(np.float32)
    m[np.arange(dst), lo] += 1.0 - frac
    m[np.arange(dst), hi] += frac
    return m


def _make_kernel(Ws, Ps, Pc, Cc, Cs, Cd, ne, nchain):
    bf16 = jnp.bfloat16
    P2 = ne * Ps
    OB_S1, OB_S2, OB_F1, OB_F2 = 0, Cs, 2 * Cs, 2 * Cs + Cd

    def gelu(x):
        h = 0.5 * x
        return h + h * jnp.tanh(0.7978845608028654 *
                                (x + 0.044715 * (x * x * x)))

    def body(cur_ref, skip_ref, m2t_ref, w1_ref, w2_ref, wr_ref, w3_ref,
             w4_ref, b_ref, out_ref):
        f32 = jnp.float32
        col = jax.lax.broadcasted_iota(jnp.int32, (1, P2), 1) % Ws
        mx_m = (col >= 1).astype(bf16)
        mx_p = (col < Ws - 1).astype(bf16)
        zrow = jnp.zeros((max(Cs, Cd), Ws), f32)

        def shift_dy(z, cout, up):
            pieces = []
            for h in range(ne):
                lo = h * Ps
                if up:
                    pieces += [zrow[0:cout], z[:, lo:lo + Ps - Ws]]
                else:
                    pieces += [z[:, lo + Ws:lo + Ps], zrow[0:cout]]
            return jnp.concatenate(pieces, axis=1)

        def conv3x3(xb, cin, wref, cout):
            t_m = jnp.concatenate([xb[:, P2 - 1:], xb[:, :P2 - 1]], axis=1) * mx_m
            t_p = jnp.concatenate([xb[:, 1:], xb[:, :1]], axis=1) * mx_p
            taps = jnp.concatenate([t_m, xb, t_p], axis=0)
            z = jnp.dot(wref[...], taps, preferred_element_type=f32)
            z_m, z_0, z_p = z[0:cout], z[cout:2 * cout], z[2 * cout:3 * cout]
            return shift_dy(z_m, cout, True) + z_0 + shift_dy(z_p, cout, False)

        def bias(off, cout):
            return b_ref[off:off + cout, :]

        for c in range(nchain):
            e0 = c * ne
            skip = jnp.concatenate([skip_ref[e0 + e] for e in range(ne)], axis=1)
            s1 = gelu(conv3x3(skip.astype(bf16), Cs, w1_ref, Cs)
                      + bias(OB_S1, Cs) + skip)
            s2 = gelu(conv3x3(s1.astype(bf16), Cs, w2_ref, Cs)
                      + bias(OB_S2, Cs) + s1)

            cur_all = cur_ref[e0:e0 + ne].reshape(ne * Cc, Pc).astype(bf16)
            cur_up = jnp.dot(cur_all, m2t_ref[...],
                             preferred_element_type=f32)
            cat = jnp.concatenate(
                [jnp.concatenate(
                    [cur_up[e * Cc:(e + 1) * Cc].astype(bf16) for e in range(ne)],
                    axis=1),
                 s2.astype(bf16)], axis=0)
            x = gelu(conv3x3(cat, Cc + Cs, wr_ref, Cd))

            f1 = gelu(conv3x3(x.astype(bf16), Cd, w3_ref, Cd)
                      + bias(OB_F1, Cd) + x)
            f2 = gelu(conv3x3(f1.astype(bf16), Cd, w4_ref, Cd)
                      + bias(OB_F2, Cd) + f1)

            for e in range(ne):
                out_ref[e0 + e] = f2[:, e * Ps:(e + 1) * Ps].astype(out_ref.dtype)

    return body


def _wall(w):
    co, ci = w.shape[3], w.shape[2]
    return jnp.transpose(w, (0, 3, 1, 2)).reshape(3 * co, 3 * ci).astype(jnp.bfloat16)


def kernel(cur_x, skip_x, skip_w1, skip_b1, skip_w2, skip_b2,
           red_w, fus_w1, fus_b1, fus_w2, fus_b2):
    N, Cc, Hc, Wc = cur_x.shape
    _, Cs, Hs, Ws = skip_x.shape
    Cd = red_w.shape[-1]
    Pc, Ps = Hc * Wc, Hs * Ws

    wh = _interp_matrix_align_corners(Hs, Hc)
    ww = _interp_matrix_align_corners(Ws, Wc)
    m2t = jnp.asarray(np.kron(wh, ww).T.astype(np.float32)).astype(jnp.bfloat16)

    b_pack = jnp.concatenate([skip_b1, skip_b2, fus_b1, fus_b2]).reshape(-1, 1)

    cur_flat = cur_x.reshape(N, Cc, Pc)
    skip_flat = skip_x.reshape(N, Cs, Ps)

    ne = 8 if N % 8 == 0 else (4 if N % 4 == 0 else (2 if N % 2 == 0 else 1))
    nchain = 1
    nblk = ne * nchain
    grid_n = N // nblk

    def const_spec(shape):
        return pl.BlockSpec(shape, lambda n: (0,) * len(shape))

    weights = [_wall(skip_w1), _wall(skip_w2), _wall(red_w),
               _wall(fus_w1), _wall(fus_w2)]

    out = pl.pallas_call(
        _make_kernel(Ws, Ps, Pc, Cc, Cs, Cd, ne, nchain),
        out_shape=jax.ShapeDtypeStruct((N, Cd, Ps), cur_x.dtype),
        grid=(grid_n,),
        in_specs=[
            pl.BlockSpec((nblk, Cc, Pc), lambda n: (n, 0, 0)),
            pl.BlockSpec((nblk, Cs, Ps), lambda n: (n, 0, 0)),
            const_spec(m2t.shape),
            const_spec(weights[0].shape),
            const_spec(weights[1].shape),
            const_spec(weights[2].shape),
            const_spec(weights[3].shape),
            const_spec(weights[4].shape),
            const_spec(b_pack.shape),
        ],
        out_specs=pl.BlockSpec((nblk, Cd, Ps), lambda n: (n, 0, 0)),
        compiler_params=pltpu.CompilerParams(
            dimension_semantics=("parallel",)),
    )(cur_flat, skip_flat, m2t, *weights, b_pack)
    return out.reshape(N, Cd, Hs, Ws)
